# baseline (speedup 1.0000x reference)
import functools

import jax
import jax.numpy as jnp
from jax import lax
from jax.experimental import pallas as pl
from jax.experimental.pallas import tpu as pltpu
from jax.experimental.pallas import tpu_sc as plsc


def _sc_gather(table3, idx2):
    f, v, d = table3.shape
    info = plsc.get_sparse_core_info()
    nw = info.num_cores * info.num_subcores
    bpw = idx2.shape[2]
    mesh = plsc.VectorSubcoreMesh(core_axis_name="c", subcore_axis_name="s")

    @functools.partial(
        pl.kernel,
        mesh=mesh,
        out_type=jax.ShapeDtypeStruct((nw, f, bpw, d), jnp.float32),
        scratch_types=[
            pltpu.VMEM((f, bpw), jnp.int32),
            pltpu.VMEM((f, bpw, d), jnp.float32),
            pltpu.SemaphoreType.DMA,
        ],
        compiler_params=pltpu.CompilerParams(use_tc_tiling_on_sc=False),
    )
    def gk(table_hbm, idx_hbm, out_hbm, idx_v, rows_v, sem):
        wid = lax.axis_index("s") * info.num_cores + lax.axis_index("c")
        pltpu.sync_copy(idx_hbm.at[wid], idx_v)
        copies = [
            pltpu.async_copy(table_hbm.at[ff].at[idx_v.at[ff]], rows_v.at[ff], sem)
            for ff in range(f)
        ]
        for c in copies:
            c.wait()
        pltpu.sync_copy(rows_v, out_hbm.at[wid])

    return gk(table3, idx2)


def _tc_interact(e2, weff, lin_tile, gmat, bias2, b, d, gb):
    f, n = e2.shape
    nb = gb * d

    def body(e_ref, w_ref, lt_ref, g_ref, b_ref, o_ref):
        e = e_ref[...]
        p = jnp.dot(w_ref[...], e, preferred_element_type=jnp.float32)
        colsum = jnp.sum(e * (p + lt_ref[...]), axis=0, keepdims=True)
        red = jnp.dot(colsum, g_ref[...], preferred_element_type=jnp.float32)
        o_ref[...] = jax.nn.sigmoid(red + b_ref[0, 0])

    return pl.pallas_call(
        body,
        grid=(n // nb,),
        in_specs=[
            pl.BlockSpec((f, nb), lambda i: (0, i)),
            pl.BlockSpec((f, f), lambda i: (0, 0)),
            pl.BlockSpec((f, nb), lambda i: (0, 0)),
            pl.BlockSpec((nb, gb), lambda i: (0, 0)),
            pl.BlockSpec((1, 1), lambda i: (0, 0)),
        ],
        out_specs=pl.BlockSpec((1, gb), lambda i: (0, i)),
        out_shape=jax.ShapeDtypeStruct((1, b), jnp.float32),
    )(e2, weff, lin_tile, gmat, bias2)


def kernel(x, emb_tables, field_cov_w, lin_w, bias):
    b, f = x.shape
    _, v, d = emb_tables.shape
    gb = 128
    nw = 32
    bpw = b // nw

    idx2 = x.astype(jnp.int32).reshape(nw, bpw, f).transpose(0, 2, 1)
    sym = (field_cov_w + field_cov_w.T) * 0.5
    weff = 0.5 * (sym - jnp.diag(jnp.diag(sym)))
    lin_tile = jnp.tile(lin_w, (1, gb))
    nb = gb * d
    gmat = (jnp.arange(nb, dtype=jnp.int32)[:, None] // d
            == jnp.arange(gb, dtype=jnp.int32)[None, :]).astype(jnp.float32)

    e4 = _sc_gather(emb_tables, idx2)
    e2 = e4.transpose(1, 0, 2, 3).reshape(f, b * d)
    out2 = _tc_interact(e2, weff, lin_tile, gmat, bias.reshape(1, 1), b, d, gb)
    return out2.reshape(b)


# speedup vs baseline: 1.9405x; 1.9405x over previous
import functools

import jax
import jax.numpy as jnp
from jax import lax
from jax.experimental import pallas as pl
from jax.experimental.pallas import tpu as pltpu
from jax.experimental.pallas import tpu_sc as plsc


def _sc_gather(table2d, idx):
    nrows = idx.shape[0]
    d = table2d.shape[1]
    info = plsc.get_sparse_core_info()
    nw = info.num_cores * info.num_subcores
    rpw = nrows // nw
    mesh = plsc.VectorSubcoreMesh(core_axis_name="c", subcore_axis_name="s")

    @functools.partial(
        pl.kernel,
        mesh=mesh,
        out_type=jax.ShapeDtypeStruct((nrows, d), jnp.float32),
        scratch_types=[
            pltpu.VMEM((rpw,), jnp.int32),
            pltpu.VMEM((rpw, d), jnp.float32),
            pltpu.SemaphoreType.DMA,
        ],
    )
    def gk(table_hbm, idx_hbm, out_hbm, idx_v, rows_v, sem):
        wid = lax.axis_index("s") * info.num_cores + lax.axis_index("c")
        base = wid * rpw
        pltpu.sync_copy(idx_hbm.at[pl.ds(base, rpw)], idx_v)

        def fire(g, carry):
            vec = idx_v[pl.ds(g * 16, 16)]
            for j in range(16):
                pltpu.async_copy(
                    table_hbm.at[pl.ds(vec[j], 1)],
                    rows_v.at[pl.ds(g * 16 + j, 1)], sem)
            return carry

        lax.fori_loop(0, rpw // 16, fire, 0)

        def drain(k, carry):
            pltpu.make_async_copy(
                table_hbm.at[pl.ds(0, 1)], rows_v.at[pl.ds(0, 1)], sem).wait()
            return carry

        lax.fori_loop(0, rpw, drain, 0)
        pltpu.sync_copy(rows_v, out_hbm.at[pl.ds(base, rpw)])

    return gk(table2d, idx)


def _tc_interact(e2, weff, lin_tile, gmat, bias2, b, d, gb):
    f, n = e2.shape
    nb = gb * d

    def body(e_ref, w_ref, lt_ref, g_ref, b_ref, o_ref):
        e = e_ref[...]
        p = jnp.dot(w_ref[...], e, preferred_element_type=jnp.float32)
        colsum = jnp.sum(e * (p + lt_ref[...]), axis=0, keepdims=True)
        red = jnp.dot(colsum, g_ref[...], preferred_element_type=jnp.float32)
        o_ref[...] = jax.nn.sigmoid(red + b_ref[0, 0])

    return pl.pallas_call(
        body,
        grid=(n // nb,),
        in_specs=[
            pl.BlockSpec((f, nb), lambda i: (0, i)),
            pl.BlockSpec((f, f), lambda i: (0, 0)),
            pl.BlockSpec((f, nb), lambda i: (0, 0)),
            pl.BlockSpec((nb, gb), lambda i: (0, 0)),
            pl.BlockSpec((1, 1), lambda i: (0, 0)),
        ],
        out_specs=pl.BlockSpec((1, gb), lambda i: (0, i)),
        out_shape=jax.ShapeDtypeStruct((1, b), jnp.float32),
    )(e2, weff, lin_tile, gmat, bias2)


def kernel(x, emb_tables, field_cov_w, lin_w, bias):
    b, f = x.shape
    _, v, d = emb_tables.shape
    gb = 128

    xt = x.T.astype(jnp.int32)
    offs = (jnp.arange(f, dtype=jnp.int32) * v)[:, None]
    idx = (xt + offs).reshape(-1)
    table2d = emb_tables.reshape(f * v, d)
    sym = (field_cov_w + field_cov_w.T) * 0.5
    weff = 0.5 * (sym - jnp.diag(jnp.diag(sym)))
    lin_tile = jnp.tile(lin_w, (1, gb))
    nb = gb * d
    gmat = (jnp.arange(nb, dtype=jnp.int32)[:, None] // d
            == jnp.arange(gb, dtype=jnp.int32)[None, :]).astype(jnp.float32)

    e_flat = _sc_gather(table2d, idx)
    e2 = e_flat.reshape(f, b * d)
    out2 = _tc_interact(e2, weff, lin_tile, gmat, bias.reshape(1, 1), b, d, gb)
    return out2.reshape(b)
